# Initial kernel scaffold; baseline (speedup 1.0000x reference)
import jax
import jax.numpy as jnp
from jax.experimental import pallas as pl


def kernel(x, edge_index, W1, b1, Wr1, br1, W2, b2, Wr2, br2, Wg, bg, Wp1, bp1, Wp2, bp2):
    raise NotImplementedError("write your pallas kernel here")


# speedup vs baseline: 7.3448x; 7.3448x over previous
import functools

import jax
import jax.numpy as jnp
from jax import lax
from jax.experimental import pallas as pl
from jax.experimental.pallas import tpu as pltpu
from jax.experimental.pallas import tpu_sc as plsc

N = 10000
E = 320000
D = 128

NC = 2
NS = 16
NW = NC * NS
EPW = E // NW
CH = 125
NCHUNK = EPW // CH
NP = 10240
RPT = NP // NS


def _seg_sum_partials(h, src3, dst3, zeros):
    mesh = plsc.VectorSubcoreMesh(core_axis_name="c", subcore_axis_name="s")

    @functools.partial(
        pl.kernel,
        out_type=jax.ShapeDtypeStruct((NC, NP, D), jnp.float32),
        mesh=mesh,
        scratch_types=[
            pltpu.VMEM((NCHUNK, CH), jnp.int32),
            pltpu.VMEM((NCHUNK, CH), jnp.int32),
            pltpu.VMEM((CH, D), jnp.float32),
            pltpu.VMEM_SHARED((NP, D), jnp.float32),
            pltpu.SemaphoreType.DMA,
        ],
    )
    def k(h_hbm, src_hbm, dst_hbm, z_hbm, out_hbm,
          src_v, dst_v, buf_a, acc, sem_a):
        cid = lax.axis_index("c")
        sid = lax.axis_index("s")
        wid = cid * NS + sid

        pltpu.sync_copy(z_hbm, acc.at[pl.ds(sid * RPT, RPT)])
        pltpu.sync_copy(src_hbm.at[wid], src_v)
        pltpu.sync_copy(dst_hbm.at[wid], dst_v)
        plsc.subcore_barrier()

        def body(j, _):
            pltpu.async_copy(h_hbm.at[src_v.at[j]], buf_a, sem_a).wait()
            pltpu.sync_copy(buf_a, acc.at[dst_v.at[j]], add=True)
            return 0

        lax.fori_loop(0, NCHUNK, body, 0)
        plsc.subcore_barrier()
        pltpu.sync_copy(acc.at[pl.ds(sid * RPT, RPT)],
                        out_hbm.at[cid, pl.ds(sid * RPT, RPT)])

    return k(h, src3, dst3, zeros)


def _elu(v):
    return jnp.where(v > 0, v, jnp.exp(jnp.minimum(v, 0.0)) - 1.0)


BN = 1000


def _dense_layer(p, h, W, b, Wr, br):
    def body(p_ref, h_ref, w_ref, b_ref, wr_ref, br_ref, o_ref):
        m = p_ref[0] + p_ref[1]
        new = _elu(jnp.dot(m, w_ref[...], preferred_element_type=jnp.float32)
                   + b_ref[...])
        res = _elu(jnp.dot(h_ref[...], wr_ref[...],
                           preferred_element_type=jnp.float32) + br_ref[...])
        o_ref[...] = new + res

    return pl.pallas_call(
        body,
        grid=(N // BN,),
        in_specs=[
            pl.BlockSpec((NC, BN, D), lambda i: (0, i, 0)),
            pl.BlockSpec((BN, D), lambda i: (i, 0)),
            pl.BlockSpec((D, D), lambda i: (0, 0)),
            pl.BlockSpec((1, D), lambda i: (0, 0)),
            pl.BlockSpec((D, D), lambda i: (0, 0)),
            pl.BlockSpec((1, D), lambda i: (0, 0)),
        ],
        out_specs=pl.BlockSpec((BN, D), lambda i: (i, 0)),
        out_shape=jax.ShapeDtypeStruct((N, D), jnp.float32),
    )(p, h, W, b, Wr, br)


def _dense2_readout(p, h, W, b, Wr, br, Wgp, bgp, Wp1, bp1, Wp2p, bp2p):
    nblk = N // BN

    def body(p_ref, h_ref, w_ref, b_ref, wr_ref, br_ref,
             wg_ref, bg_ref, wp1_ref, bp1_ref, wp2_ref, bp2_ref,
             o_ref, s_acc, m_acc):
        i = pl.program_id(0)
        m = p_ref[0] + p_ref[1]
        new = _elu(jnp.dot(m, w_ref[...], preferred_element_type=jnp.float32)
                   + b_ref[...])
        res = _elu(jnp.dot(h_ref[...], wr_ref[...],
                           preferred_element_type=jnp.float32) + br_ref[...])
        h2 = new + res
        gate = jnp.dot(h2, wg_ref[...], preferred_element_type=jnp.float32) \
            + bg_ref[...]
        w = 1.0 / (1.0 + jnp.exp(-gate[:, 0:1]))
        blk_sum = jnp.sum(w * h2, axis=0, keepdims=True)
        blk_max = jnp.max(h2, axis=0, keepdims=True)

        @pl.when(i == 0)
        def _():
            s_acc[...] = blk_sum
            m_acc[...] = blk_max

        @pl.when(i > 0)
        def _():
            s_acc[...] = s_acc[...] + blk_sum
            m_acc[...] = jnp.maximum(m_acc[...], blk_max)

        @pl.when(i == nblk - 1)
        def _():
            g = jnp.concatenate([s_acc[...], m_acc[...]], axis=1)
            z = jnp.maximum(
                jnp.dot(g, wp1_ref[...], preferred_element_type=jnp.float32)
                + bp1_ref[...], 0.0)
            z = z * (1.0 / jnp.sqrt(1.0 + 1e-5))
            o_ref[...] = jnp.dot(z, wp2_ref[...],
                                 preferred_element_type=jnp.float32) \
                + bp2_ref[...]

    return pl.pallas_call(
        body,
        grid=(nblk,),
        in_specs=[
            pl.BlockSpec((NC, BN, D), lambda i: (0, i, 0)),
            pl.BlockSpec((BN, D), lambda i: (i, 0)),
            pl.BlockSpec((D, D), lambda i: (0, 0)),
            pl.BlockSpec((1, D), lambda i: (0, 0)),
            pl.BlockSpec((D, D), lambda i: (0, 0)),
            pl.BlockSpec((1, D), lambda i: (0, 0)),
            pl.BlockSpec((D, D), lambda i: (0, 0)),
            pl.BlockSpec((1, D), lambda i: (0, 0)),
            pl.BlockSpec((2 * D, D), lambda i: (0, 0)),
            pl.BlockSpec((1, D), lambda i: (0, 0)),
            pl.BlockSpec((D, D), lambda i: (0, 0)),
            pl.BlockSpec((1, D), lambda i: (0, 0)),
        ],
        out_specs=pl.BlockSpec((1, D), lambda i: (0, 0)),
        out_shape=jax.ShapeDtypeStruct((1, D), jnp.float32),
        scratch_shapes=[
            pltpu.VMEM((1, D), jnp.float32),
            pltpu.VMEM((1, D), jnp.float32),
        ],
    )(p, h, W, b, Wr, br, Wgp, bgp, Wp1, bp1, Wp2p, bp2p)


def kernel(x, edge_index, W1, b1, Wr1, br1, W2, b2, Wr2, br2,
           Wg, bg, Wp1, bp1, Wp2, bp2):
    src3 = edge_index[0].reshape(NW, NCHUNK, CH)
    dst3 = edge_index[1].reshape(NW, NCHUNK, CH)
    zeros = jnp.zeros((RPT, D), jnp.float32)

    Wgp = jnp.pad(Wg, ((0, 0), (0, D - Wg.shape[1])))
    bgp = jnp.pad(bg, (0, D - bg.shape[0])).reshape(1, D)
    Wp2p = jnp.pad(Wp2, ((0, 0), (0, D - Wp2.shape[1])))
    bp2p = jnp.pad(bp2, (0, D - bp2.shape[0])).reshape(1, D)

    b1r = b1.reshape(1, D)
    br1r = br1.reshape(1, D)
    b2r = b2.reshape(1, D)
    br2r = br2.reshape(1, D)
    bp1r = bp1.reshape(1, D)

    p1 = _seg_sum_partials(x, src3, dst3, zeros)[:, :N]
    h1 = _dense_layer(p1, x, W1, b1r, Wr1, br1r)
    p2 = _seg_sum_partials(h1, src3, dst3, zeros)[:, :N]
    out = _dense2_readout(p2, h1, W2, b2r, Wr2, br2r,
                          Wgp, bgp, Wp1, bp1r, Wp2p, bp2p)
    return out[:, :12]


# speedup vs baseline: 9.2448x; 1.2587x over previous
import functools

import jax
import jax.numpy as jnp
from jax import lax
from jax.experimental import pallas as pl
from jax.experimental.pallas import tpu as pltpu
from jax.experimental.pallas import tpu_sc as plsc

N = 10000
E = 320000
D = 128

NC = 2
NS = 16
NW = NC * NS
EPW = E // NW
CH = 125
NCHUNK = EPW // CH
HALF = NCHUNK // 2
NP = 10240
RPT = NP // NS


def _seg_sum_partials(h, src3, dst3, zeros):
    mesh = plsc.VectorSubcoreMesh(core_axis_name="c", subcore_axis_name="s")

    @functools.partial(
        pl.kernel,
        out_type=jax.ShapeDtypeStruct((NC, NP, D), jnp.float32),
        mesh=mesh,
        scratch_types=[
            pltpu.VMEM((HALF, CH), jnp.int32),
            pltpu.VMEM((HALF, CH), jnp.int32),
            pltpu.VMEM((CH, D), jnp.float32),
            pltpu.VMEM((CH, D), jnp.float32),
            pltpu.VMEM_SHARED((NP, D), jnp.float32),
            pltpu.SemaphoreType.DMA,
            pltpu.SemaphoreType.DMA,
            pltpu.SemaphoreType.DMA,
            pltpu.SemaphoreType.DMA,
        ],
    )
    def k(h_hbm, src_hbm, dst_hbm, z_hbm, out_hbm,
          src_v, dst_v, buf_a, buf_b, acc, sem_ga, sem_gb, sem_sa, sem_sb):
        cid = lax.axis_index("c")
        sid = lax.axis_index("s")
        wid = cid * NS + sid

        pltpu.sync_copy(z_hbm, acc.at[pl.ds(sid * RPT, RPT)])
        plsc.subcore_barrier()

        def step(j, buf, sem_g, sem_s, prev_buf, prev_sem_g, prev_sem_s):
            pltpu.make_async_copy(h_hbm.at[src_v.at[j]], buf, sem_g).wait()
            pltpu.async_copy(buf, acc.at[dst_v.at[j]], sem_s, add=True)

            @pl.when(j >= 1)
            def _():
                pltpu.make_async_copy(
                    prev_buf, acc.at[dst_v.at[j - 1]], prev_sem_s).wait()

            @pl.when(j + 1 < HALF)
            def _():
                pltpu.async_copy(h_hbm.at[src_v.at[j + 1]], prev_buf,
                                 prev_sem_g)

        for p in range(NCHUNK // HALF):
            pltpu.sync_copy(src_hbm.at[wid, pl.ds(p * HALF, HALF)], src_v)
            pltpu.sync_copy(dst_hbm.at[wid, pl.ds(p * HALF, HALF)], dst_v)
            pltpu.async_copy(h_hbm.at[src_v.at[0]], buf_a, sem_ga)

            def body(j2, _):
                step(2 * j2, buf_a, sem_ga, sem_sa, buf_b, sem_gb, sem_sb)
                step(2 * j2 + 1, buf_b, sem_gb, sem_sb, buf_a, sem_ga, sem_sa)
                return 0

            lax.fori_loop(0, HALF // 2, body, 0)
            pltpu.make_async_copy(
                buf_b, acc.at[dst_v.at[HALF - 1]], sem_sb).wait()
        plsc.subcore_barrier()
        pltpu.sync_copy(acc.at[pl.ds(sid * RPT, RPT)],
                        out_hbm.at[cid, pl.ds(sid * RPT, RPT)])

    return k(h, src3, dst3, zeros)


def _elu(v):
    return jnp.where(v > 0, v, jnp.exp(jnp.minimum(v, 0.0)) - 1.0)


BN = 1000


def _dense_layer(p, h, W, b, Wr, br):
    def body(p_ref, h_ref, w_ref, b_ref, wr_ref, br_ref, o_ref):
        m = p_ref[0] + p_ref[1]
        new = _elu(jnp.dot(m, w_ref[...], preferred_element_type=jnp.float32)
                   + b_ref[...])
        res = _elu(jnp.dot(h_ref[...], wr_ref[...],
                           preferred_element_type=jnp.float32) + br_ref[...])
        o_ref[...] = new + res

    return pl.pallas_call(
        body,
        grid=(N // BN,),
        in_specs=[
            pl.BlockSpec((NC, BN, D), lambda i: (0, i, 0)),
            pl.BlockSpec((BN, D), lambda i: (i, 0)),
            pl.BlockSpec((D, D), lambda i: (0, 0)),
            pl.BlockSpec((1, D), lambda i: (0, 0)),
            pl.BlockSpec((D, D), lambda i: (0, 0)),
            pl.BlockSpec((1, D), lambda i: (0, 0)),
        ],
        out_specs=pl.BlockSpec((BN, D), lambda i: (i, 0)),
        out_shape=jax.ShapeDtypeStruct((N, D), jnp.float32),
    )(p, h, W, b, Wr, br)


def _dense2_readout(p, h, W, b, Wr, br, Wgp, bgp, Wp1, bp1, Wp2p, bp2p):
    nblk = N // BN

    def body(p_ref, h_ref, w_ref, b_ref, wr_ref, br_ref,
             wg_ref, bg_ref, wp1_ref, bp1_ref, wp2_ref, bp2_ref,
             o_ref, s_acc, m_acc):
        i = pl.program_id(0)
        m = p_ref[0] + p_ref[1]
        new = _elu(jnp.dot(m, w_ref[...], preferred_element_type=jnp.float32)
                   + b_ref[...])
        res = _elu(jnp.dot(h_ref[...], wr_ref[...],
                           preferred_element_type=jnp.float32) + br_ref[...])
        h2 = new + res
        gate = jnp.dot(h2, wg_ref[...], preferred_element_type=jnp.float32) \
            + bg_ref[...]
        w = 1.0 / (1.0 + jnp.exp(-gate[:, 0:1]))
        blk_sum = jnp.sum(w * h2, axis=0, keepdims=True)
        blk_max = jnp.max(h2, axis=0, keepdims=True)

        @pl.when(i == 0)
        def _():
            s_acc[...] = blk_sum
            m_acc[...] = blk_max

        @pl.when(i > 0)
        def _():
            s_acc[...] = s_acc[...] + blk_sum
            m_acc[...] = jnp.maximum(m_acc[...], blk_max)

        @pl.when(i == nblk - 1)
        def _():
            g = jnp.concatenate([s_acc[...], m_acc[...]], axis=1)
            z = jnp.maximum(
                jnp.dot(g, wp1_ref[...], preferred_element_type=jnp.float32)
                + bp1_ref[...], 0.0)
            z = z * (1.0 / jnp.sqrt(1.0 + 1e-5))
            o_ref[...] = jnp.dot(z, wp2_ref[...],
                                 preferred_element_type=jnp.float32) \
                + bp2_ref[...]

    return pl.pallas_call(
        body,
        grid=(nblk,),
        in_specs=[
            pl.BlockSpec((NC, BN, D), lambda i: (0, i, 0)),
            pl.BlockSpec((BN, D), lambda i: (i, 0)),
            pl.BlockSpec((D, D), lambda i: (0, 0)),
            pl.BlockSpec((1, D), lambda i: (0, 0)),
            pl.BlockSpec((D, D), lambda i: (0, 0)),
            pl.BlockSpec((1, D), lambda i: (0, 0)),
            pl.BlockSpec((D, D), lambda i: (0, 0)),
            pl.BlockSpec((1, D), lambda i: (0, 0)),
            pl.BlockSpec((2 * D, D), lambda i: (0, 0)),
            pl.BlockSpec((1, D), lambda i: (0, 0)),
            pl.BlockSpec((D, D), lambda i: (0, 0)),
            pl.BlockSpec((1, D), lambda i: (0, 0)),
        ],
        out_specs=pl.BlockSpec((1, D), lambda i: (0, 0)),
        out_shape=jax.ShapeDtypeStruct((1, D), jnp.float32),
        scratch_shapes=[
            pltpu.VMEM((1, D), jnp.float32),
            pltpu.VMEM((1, D), jnp.float32),
        ],
    )(p, h, W, b, Wr, br, Wgp, bgp, Wp1, bp1, Wp2p, bp2p)


def kernel(x, edge_index, W1, b1, Wr1, br1, W2, b2, Wr2, br2,
           Wg, bg, Wp1, bp1, Wp2, bp2):
    src3 = edge_index[0].reshape(NW, NCHUNK, CH)
    dst3 = edge_index[1].reshape(NW, NCHUNK, CH)
    zeros = jnp.zeros((RPT, D), jnp.float32)

    Wgp = jnp.pad(Wg, ((0, 0), (0, D - Wg.shape[1])))
    bgp = jnp.pad(bg, (0, D - bg.shape[0])).reshape(1, D)
    Wp2p = jnp.pad(Wp2, ((0, 0), (0, D - Wp2.shape[1])))
    bp2p = jnp.pad(bp2, (0, D - bp2.shape[0])).reshape(1, D)

    b1r = b1.reshape(1, D)
    br1r = br1.reshape(1, D)
    b2r = b2.reshape(1, D)
    br2r = br2.reshape(1, D)
    bp1r = bp1.reshape(1, D)

    p1 = _seg_sum_partials(x, src3, dst3, zeros)[:, :N]
    h1 = _dense_layer(p1, x, W1, b1r, Wr1, br1r)
    p2 = _seg_sum_partials(h1, src3, dst3, zeros)[:, :N]
    out = _dense2_readout(p2, h1, W2, b2r, Wr2, br2r,
                          Wgp, bgp, Wp1, bp1r, Wp2p, bp2p)
    return out[:, :12]


# speedup vs baseline: 9.2844x; 1.0043x over previous
import functools

import jax
import jax.numpy as jnp
from jax import lax
from jax.experimental import pallas as pl
from jax.experimental.pallas import tpu as pltpu
from jax.experimental.pallas import tpu_sc as plsc

N = 10000
E = 320000
D = 128

NC = 2
NS = 16
NW = NC * NS
EPW = E // NW
CH = 50
NCHUNK = EPW // CH
SCH = 40
NSTAGE = NCHUNK // SCH
KSLOT = 4
NP = 10240
RPT = NP // NS


def _seg_sum_partials(h, src3, dst3, zeros):
    mesh = plsc.VectorSubcoreMesh(core_axis_name="c", subcore_axis_name="s")

    @functools.partial(
        pl.kernel,
        out_type=jax.ShapeDtypeStruct((NC, NP, D), jnp.float32),
        mesh=mesh,
        scratch_types=[
            pltpu.VMEM((SCH, CH), jnp.int32),
            pltpu.VMEM((SCH, CH), jnp.int32),
            [pltpu.VMEM((CH, D), jnp.float32) for _ in range(KSLOT)],
            [pltpu.SemaphoreType.DMA for _ in range(KSLOT)],
            [pltpu.SemaphoreType.DMA for _ in range(KSLOT)],
            pltpu.VMEM_SHARED((NP, D), jnp.float32),
        ],
    )
    def k(h_hbm, src_hbm, dst_hbm, z_hbm, out_hbm,
          src_v, dst_v, bufs, gsems, ssems, acc):
        cid = lax.axis_index("c")
        sid = lax.axis_index("s")
        wid = cid * NS + sid

        pltpu.sync_copy(z_hbm, acc.at[pl.ds(sid * RPT, RPT)])
        plsc.subcore_barrier()

        def gather(j, s):
            pltpu.async_copy(h_hbm.at[src_v.at[j]], bufs[s], gsems[s])

        def step(j, b):
            s = b % KSLOT
            pltpu.make_async_copy(h_hbm.at[src_v.at[j]], bufs[s],
                                  gsems[s]).wait()
            pltpu.async_copy(bufs[s], acc.at[dst_v.at[j]], ssems[s], add=True)

            sp = (b - 2) % KSLOT

            @pl.when(j >= 2)
            def _():
                pltpu.make_async_copy(bufs[sp], acc.at[dst_v.at[j - 2]],
                                      ssems[sp]).wait()

            @pl.when(j + 2 < SCH)
            def _():
                gather(j + 2, (b + 2) % KSLOT)

        for p in range(NSTAGE):
            pltpu.sync_copy(src_hbm.at[wid, p], src_v)
            pltpu.sync_copy(dst_hbm.at[wid, p], dst_v)
            gather(0, 0)
            gather(1, 1)

            def body(t, _):
                for b in range(KSLOT):
                    step(KSLOT * t + b, b)
                return 0

            lax.fori_loop(0, SCH // KSLOT, body, 0)
            pltpu.make_async_copy(bufs[(SCH - 2) % KSLOT],
                                  acc.at[dst_v.at[SCH - 2]],
                                  ssems[(SCH - 2) % KSLOT]).wait()
            pltpu.make_async_copy(bufs[(SCH - 1) % KSLOT],
                                  acc.at[dst_v.at[SCH - 1]],
                                  ssems[(SCH - 1) % KSLOT]).wait()
        plsc.subcore_barrier()
        pltpu.sync_copy(acc.at[pl.ds(sid * RPT, RPT)],
                        out_hbm.at[cid, pl.ds(sid * RPT, RPT)])

    return k(h, src3, dst3, zeros)


def _elu(v):
    return jnp.where(v > 0, v, jnp.exp(jnp.minimum(v, 0.0)) - 1.0)


BN = 1000


def _dense_layer(p, h, W, b, Wr, br):
    def body(p_ref, h_ref, w_ref, b_ref, wr_ref, br_ref, o_ref):
        m = p_ref[0] + p_ref[1]
        new = _elu(jnp.dot(m, w_ref[...], preferred_element_type=jnp.float32)
                   + b_ref[...])
        res = _elu(jnp.dot(h_ref[...], wr_ref[...],
                           preferred_element_type=jnp.float32) + br_ref[...])
        o_ref[...] = new + res

    return pl.pallas_call(
        body,
        grid=(N // BN,),
        in_specs=[
            pl.BlockSpec((NC, BN, D), lambda i: (0, i, 0)),
            pl.BlockSpec((BN, D), lambda i: (i, 0)),
            pl.BlockSpec((D, D), lambda i: (0, 0)),
            pl.BlockSpec((1, D), lambda i: (0, 0)),
            pl.BlockSpec((D, D), lambda i: (0, 0)),
            pl.BlockSpec((1, D), lambda i: (0, 0)),
        ],
        out_specs=pl.BlockSpec((BN, D), lambda i: (i, 0)),
        out_shape=jax.ShapeDtypeStruct((N, D), jnp.float32),
    )(p, h, W, b, Wr, br)


def _dense2_readout(p, h, W, b, Wr, br, Wgp, bgp, Wp1, bp1, Wp2p, bp2p):
    nblk = N // BN

    def body(p_ref, h_ref, w_ref, b_ref, wr_ref, br_ref,
             wg_ref, bg_ref, wp1_ref, bp1_ref, wp2_ref, bp2_ref,
             o_ref, s_acc, m_acc):
        i = pl.program_id(0)
        m = p_ref[0] + p_ref[1]
        new = _elu(jnp.dot(m, w_ref[...], preferred_element_type=jnp.float32)
                   + b_ref[...])
        res = _elu(jnp.dot(h_ref[...], wr_ref[...],
                           preferred_element_type=jnp.float32) + br_ref[...])
        h2 = new + res
        gate = jnp.dot(h2, wg_ref[...], preferred_element_type=jnp.float32) \
            + bg_ref[...]
        w = 1.0 / (1.0 + jnp.exp(-gate[:, 0:1]))
        blk_sum = jnp.sum(w * h2, axis=0, keepdims=True)
        blk_max = jnp.max(h2, axis=0, keepdims=True)

        @pl.when(i == 0)
        def _():
            s_acc[...] = blk_sum
            m_acc[...] = blk_max

        @pl.when(i > 0)
        def _():
            s_acc[...] = s_acc[...] + blk_sum
            m_acc[...] = jnp.maximum(m_acc[...], blk_max)

        @pl.when(i == nblk - 1)
        def _():
            g = jnp.concatenate([s_acc[...], m_acc[...]], axis=1)
            z = jnp.maximum(
                jnp.dot(g, wp1_ref[...], preferred_element_type=jnp.float32)
                + bp1_ref[...], 0.0)
            z = z * (1.0 / jnp.sqrt(1.0 + 1e-5))
            o_ref[...] = jnp.dot(z, wp2_ref[...],
                                 preferred_element_type=jnp.float32) \
                + bp2_ref[...]

    return pl.pallas_call(
        body,
        grid=(nblk,),
        in_specs=[
            pl.BlockSpec((NC, BN, D), lambda i: (0, i, 0)),
            pl.BlockSpec((BN, D), lambda i: (i, 0)),
            pl.BlockSpec((D, D), lambda i: (0, 0)),
            pl.BlockSpec((1, D), lambda i: (0, 0)),
            pl.BlockSpec((D, D), lambda i: (0, 0)),
            pl.BlockSpec((1, D), lambda i: (0, 0)),
            pl.BlockSpec((D, D), lambda i: (0, 0)),
            pl.BlockSpec((1, D), lambda i: (0, 0)),
            pl.BlockSpec((2 * D, D), lambda i: (0, 0)),
            pl.BlockSpec((1, D), lambda i: (0, 0)),
            pl.BlockSpec((D, D), lambda i: (0, 0)),
            pl.BlockSpec((1, D), lambda i: (0, 0)),
        ],
        out_specs=pl.BlockSpec((1, D), lambda i: (0, 0)),
        out_shape=jax.ShapeDtypeStruct((1, D), jnp.float32),
        scratch_shapes=[
            pltpu.VMEM((1, D), jnp.float32),
            pltpu.VMEM((1, D), jnp.float32),
        ],
    )(p, h, W, b, Wr, br, Wgp, bgp, Wp1, bp1, Wp2p, bp2p)


def kernel(x, edge_index, W1, b1, Wr1, br1, W2, b2, Wr2, br2,
           Wg, bg, Wp1, bp1, Wp2, bp2):
    src3 = edge_index[0].reshape(NW, NSTAGE, SCH, CH)
    dst3 = edge_index[1].reshape(NW, NSTAGE, SCH, CH)
    zeros = jnp.zeros((RPT, D), jnp.float32)

    Wgp = jnp.pad(Wg, ((0, 0), (0, D - Wg.shape[1])))
    bgp = jnp.pad(bg, (0, D - bg.shape[0])).reshape(1, D)
    Wp2p = jnp.pad(Wp2, ((0, 0), (0, D - Wp2.shape[1])))
    bp2p = jnp.pad(bp2, (0, D - bp2.shape[0])).reshape(1, D)

    b1r = b1.reshape(1, D)
    br1r = br1.reshape(1, D)
    b2r = b2.reshape(1, D)
    br2r = br2.reshape(1, D)
    bp1r = bp1.reshape(1, D)

    p1 = _seg_sum_partials(x, src3, dst3, zeros)
    h1 = _dense_layer(p1, x, W1, b1r, Wr1, br1r)
    p2 = _seg_sum_partials(h1, src3, dst3, zeros)
    out = _dense2_readout(p2, h1, W2, b2r, Wr2, br2r,
                          Wgp, bgp, Wp1, bp1r, Wp2p, bp2p)
    return out[:, :12]


# speedup vs baseline: 9.7546x; 1.0506x over previous
import functools

import jax
import jax.numpy as jnp
from jax import lax
from jax.experimental import pallas as pl
from jax.experimental.pallas import tpu as pltpu
from jax.experimental.pallas import tpu_sc as plsc

N = 10000
E = 320000
D = 128

NC = 2
NS = 16
NW = NC * NS
EPW = E // NW
CH = 125
NCHUNK = EPW // CH
HALF = NCHUNK // 2
NP = 10240
RPT = NP // NS


def _seg_sum_partials(h, src3, dst3, zeros):
    mesh = plsc.VectorSubcoreMesh(core_axis_name="c", subcore_axis_name="s")

    @functools.partial(
        pl.kernel,
        out_type=jax.ShapeDtypeStruct((NC, NP, D), jnp.float32),
        mesh=mesh,
        scratch_types=[
            pltpu.VMEM((HALF, CH), jnp.int32),
            pltpu.VMEM((HALF, CH), jnp.int32),
            pltpu.VMEM((CH, D), jnp.float32),
            pltpu.VMEM((CH, D), jnp.float32),
            pltpu.VMEM_SHARED((NP, D), jnp.float32),
            pltpu.SemaphoreType.DMA,
            pltpu.SemaphoreType.DMA,
            pltpu.SemaphoreType.DMA,
            pltpu.SemaphoreType.DMA,
        ],
    )
    def k(h_hbm, src_hbm, dst_hbm, z_hbm, out_hbm,
          src_v, dst_v, buf_a, buf_b, acc, sem_ga, sem_gb, sem_sa, sem_sb):
        cid = lax.axis_index("c")
        sid = lax.axis_index("s")
        wid = cid * NS + sid

        pltpu.sync_copy(z_hbm.at[pl.ds(sid * RPT, RPT)],
                        acc.at[pl.ds(sid * RPT, RPT)])
        plsc.subcore_barrier()

        def step(j, buf, sem_g, sem_s, prev_buf, prev_sem_g, prev_sem_s):
            pltpu.make_async_copy(h_hbm.at[src_v.at[j]], buf, sem_g).wait()
            pltpu.async_copy(buf, acc.at[dst_v.at[j]], sem_s, add=True)

            @pl.when(j >= 1)
            def _():
                pltpu.make_async_copy(
                    prev_buf, acc.at[dst_v.at[j - 1]], prev_sem_s).wait()

            @pl.when(j + 1 < HALF)
            def _():
                pltpu.async_copy(h_hbm.at[src_v.at[j + 1]], prev_buf,
                                 prev_sem_g)

        for p in range(NCHUNK // HALF):
            pltpu.sync_copy(src_hbm.at[wid, pl.ds(p * HALF, HALF)], src_v)
            pltpu.sync_copy(dst_hbm.at[wid, pl.ds(p * HALF, HALF)], dst_v)
            pltpu.async_copy(h_hbm.at[src_v.at[0]], buf_a, sem_ga)

            def body(j2, _):
                step(2 * j2, buf_a, sem_ga, sem_sa, buf_b, sem_gb, sem_sb)
                step(2 * j2 + 1, buf_b, sem_gb, sem_sb, buf_a, sem_ga, sem_sa)
                return 0

            lax.fori_loop(0, HALF // 2, body, 0)
            pltpu.make_async_copy(
                buf_b, acc.at[dst_v.at[HALF - 1]], sem_sb).wait()
        plsc.subcore_barrier()
        pltpu.sync_copy(acc.at[pl.ds(sid * RPT, RPT)],
                        out_hbm.at[cid, pl.ds(sid * RPT, RPT)])

    return k(h, src3, dst3, zeros)


def _elu(v):
    return jnp.where(v > 0, v, jnp.exp(jnp.minimum(v, 0.0)) - 1.0)


BN = 1000


def _dense_layer(p, h, W, b, Wr, br):
    def body(p_ref, h_ref, w_ref, b_ref, wr_ref, br_ref, o_ref):
        m = p_ref[0] + p_ref[1]
        new = _elu(jnp.dot(m, w_ref[...], preferred_element_type=jnp.float32)
                   + b_ref[...])
        res = _elu(jnp.dot(h_ref[...], wr_ref[...],
                           preferred_element_type=jnp.float32) + br_ref[...])
        o_ref[...] = new + res

    return pl.pallas_call(
        body,
        grid=(N // BN,),
        in_specs=[
            pl.BlockSpec((NC, BN, D), lambda i: (0, i, 0)),
            pl.BlockSpec((BN, D), lambda i: (i, 0)),
            pl.BlockSpec((D, D), lambda i: (0, 0)),
            pl.BlockSpec((1, D), lambda i: (0, 0)),
            pl.BlockSpec((D, D), lambda i: (0, 0)),
            pl.BlockSpec((1, D), lambda i: (0, 0)),
        ],
        out_specs=pl.BlockSpec((BN, D), lambda i: (i, 0)),
        out_shape=jax.ShapeDtypeStruct((N, D), jnp.float32),
    )(p, h, W, b, Wr, br)


def _dense2_readout(p, h, W, b, Wr, br, Wgp, bgp, Wp1, bp1, Wp2p, bp2p):
    nblk = N // BN

    def body(p_ref, h_ref, w_ref, b_ref, wr_ref, br_ref,
             wg_ref, bg_ref, wp1_ref, bp1_ref, wp2_ref, bp2_ref,
             o_ref, s_acc, m_acc):
        i = pl.program_id(0)
        m = p_ref[0] + p_ref[1]
        new = _elu(jnp.dot(m, w_ref[...], preferred_element_type=jnp.float32)
                   + b_ref[...])
        res = _elu(jnp.dot(h_ref[...], wr_ref[...],
                           preferred_element_type=jnp.float32) + br_ref[...])
        h2 = new + res
        gate = jnp.dot(h2, wg_ref[...], preferred_element_type=jnp.float32) \
            + bg_ref[...]
        w = 1.0 / (1.0 + jnp.exp(-gate[:, 0:1]))
        blk_sum = jnp.sum(w * h2, axis=0, keepdims=True)
        blk_max = jnp.max(h2, axis=0, keepdims=True)

        @pl.when(i == 0)
        def _():
            s_acc[...] = blk_sum
            m_acc[...] = blk_max

        @pl.when(i > 0)
        def _():
            s_acc[...] = s_acc[...] + blk_sum
            m_acc[...] = jnp.maximum(m_acc[...], blk_max)

        @pl.when(i == nblk - 1)
        def _():
            g = jnp.concatenate([s_acc[...], m_acc[...]], axis=1)
            z = jnp.maximum(
                jnp.dot(g, wp1_ref[...], preferred_element_type=jnp.float32)
                + bp1_ref[...], 0.0)
            z = z * (1.0 / jnp.sqrt(1.0 + 1e-5))
            o_ref[...] = jnp.dot(z, wp2_ref[...],
                                 preferred_element_type=jnp.float32) \
                + bp2_ref[...]

    return pl.pallas_call(
        body,
        grid=(nblk,),
        in_specs=[
            pl.BlockSpec((NC, BN, D), lambda i: (0, i, 0)),
            pl.BlockSpec((BN, D), lambda i: (i, 0)),
            pl.BlockSpec((D, D), lambda i: (0, 0)),
            pl.BlockSpec((1, D), lambda i: (0, 0)),
            pl.BlockSpec((D, D), lambda i: (0, 0)),
            pl.BlockSpec((1, D), lambda i: (0, 0)),
            pl.BlockSpec((D, D), lambda i: (0, 0)),
            pl.BlockSpec((1, D), lambda i: (0, 0)),
            pl.BlockSpec((2 * D, D), lambda i: (0, 0)),
            pl.BlockSpec((1, D), lambda i: (0, 0)),
            pl.BlockSpec((D, D), lambda i: (0, 0)),
            pl.BlockSpec((1, D), lambda i: (0, 0)),
        ],
        out_specs=pl.BlockSpec((1, D), lambda i: (0, 0)),
        out_shape=jax.ShapeDtypeStruct((1, D), jnp.float32),
        scratch_shapes=[
            pltpu.VMEM((1, D), jnp.float32),
            pltpu.VMEM((1, D), jnp.float32),
        ],
    )(p, h, W, b, Wr, br, Wgp, bgp, Wp1, bp1, Wp2p, bp2p)


def kernel(x, edge_index, W1, b1, Wr1, br1, W2, b2, Wr2, br2,
           Wg, bg, Wp1, bp1, Wp2, bp2):
    src3 = edge_index[0].reshape(NW, NCHUNK, CH)
    dst3 = edge_index[1].reshape(NW, NCHUNK, CH)
    zeros = jnp.zeros((NP, D), jnp.float32)

    Wgp = jnp.pad(Wg, ((0, 0), (0, D - Wg.shape[1])))
    bgp = jnp.pad(bg, (0, D - bg.shape[0])).reshape(1, D)
    Wp2p = jnp.pad(Wp2, ((0, 0), (0, D - Wp2.shape[1])))
    bp2p = jnp.pad(bp2, (0, D - bp2.shape[0])).reshape(1, D)

    b1r = b1.reshape(1, D)
    br1r = br1.reshape(1, D)
    b2r = b2.reshape(1, D)
    br2r = br2.reshape(1, D)
    bp1r = bp1.reshape(1, D)

    p1 = _seg_sum_partials(x, src3, dst3, zeros)
    h1 = _dense_layer(p1, x, W1, b1r, Wr1, br1r)
    p2 = _seg_sum_partials(h1, src3, dst3, zeros)
    out = _dense2_readout(p2, h1, W2, b2r, Wr2, br2r,
                          Wgp, bgp, Wp1, bp1r, Wp2p, bp2p)
    return out[:, :12]


# speedup vs baseline: 9.7736x; 1.0020x over previous
import functools

import jax
import jax.numpy as jnp
from jax import lax
from jax.experimental import pallas as pl
from jax.experimental.pallas import tpu as pltpu
from jax.experimental.pallas import tpu_sc as plsc

N = 10000
E = 320000
D = 128

NC = 2
NS = 16
NW = NC * NS
EPW = E // NW
CH = 125
NCHUNK = EPW // CH
HALF = NCHUNK // 2
NP = 10240
RPT = NP // NS


def _seg_sum_partials(h, src3, dst3, zeros):
    mesh = plsc.VectorSubcoreMesh(core_axis_name="c", subcore_axis_name="s")

    @functools.partial(
        pl.kernel,
        out_type=jax.ShapeDtypeStruct((NC, NP, D), jnp.float32),
        mesh=mesh,
        scratch_types=[
            pltpu.VMEM((HALF, CH), jnp.int32),
            pltpu.VMEM((HALF, CH), jnp.int32),
            pltpu.VMEM((CH, D), jnp.float32),
            pltpu.VMEM((CH, D), jnp.float32),
            pltpu.VMEM_SHARED((NP, D), jnp.float32),
            pltpu.SemaphoreType.DMA,
            pltpu.SemaphoreType.DMA,
            pltpu.SemaphoreType.DMA,
            pltpu.SemaphoreType.DMA,
        ],
    )
    def k(h_hbm, src_hbm, dst_hbm, z_hbm, out_hbm,
          src_v, dst_v, buf_a, buf_b, acc, sem_ga, sem_gb, sem_sa, sem_sb):
        cid = lax.axis_index("c")
        sid = lax.axis_index("s")
        wid = cid * NS + sid

        pltpu.sync_copy(z_hbm.at[pl.ds(sid * RPT, RPT)],
                        acc.at[pl.ds(sid * RPT, RPT)])
        plsc.subcore_barrier()

        def step(j, buf, sem_g, sem_s, prev_buf, prev_sem_g, prev_sem_s):
            pltpu.make_async_copy(h_hbm.at[src_v.at[j]], buf, sem_g).wait()

            @pl.when(j + 1 < HALF)
            def _():
                pltpu.async_copy(h_hbm.at[src_v.at[j + 1]], prev_buf,
                                 prev_sem_g)

            pltpu.async_copy(buf, acc.at[dst_v.at[j]], sem_s, add=True).wait()

        for p in range(NCHUNK // HALF):
            pltpu.sync_copy(src_hbm.at[wid, pl.ds(p * HALF, HALF)], src_v)
            pltpu.sync_copy(dst_hbm.at[wid, pl.ds(p * HALF, HALF)], dst_v)
            pltpu.async_copy(h_hbm.at[src_v.at[0]], buf_a, sem_ga)

            def body(j2, _):
                step(2 * j2, buf_a, sem_ga, sem_sa, buf_b, sem_gb, sem_sb)
                step(2 * j2 + 1, buf_b, sem_gb, sem_sb, buf_a, sem_ga, sem_sa)
                return 0

            lax.fori_loop(0, HALF // 2, body, 0)
        plsc.subcore_barrier()
        pltpu.sync_copy(acc.at[pl.ds(sid * RPT, RPT)],
                        out_hbm.at[cid, pl.ds(sid * RPT, RPT)])

    return k(h, src3, dst3, zeros)


def _elu(v):
    return jnp.where(v > 0, v, jnp.exp(jnp.minimum(v, 0.0)) - 1.0)


BN = 1000


def _dense_layer(p, h, W, b, Wr, br):
    def body(p_ref, h_ref, w_ref, b_ref, wr_ref, br_ref, o_ref):
        m = p_ref[0] + p_ref[1]
        new = _elu(jnp.dot(m, w_ref[...], preferred_element_type=jnp.float32)
                   + b_ref[...])
        res = _elu(jnp.dot(h_ref[...], wr_ref[...],
                           preferred_element_type=jnp.float32) + br_ref[...])
        o_ref[...] = new + res

    return pl.pallas_call(
        body,
        grid=(N // BN,),
        in_specs=[
            pl.BlockSpec((NC, BN, D), lambda i: (0, i, 0)),
            pl.BlockSpec((BN, D), lambda i: (i, 0)),
            pl.BlockSpec((D, D), lambda i: (0, 0)),
            pl.BlockSpec((1, D), lambda i: (0, 0)),
            pl.BlockSpec((D, D), lambda i: (0, 0)),
            pl.BlockSpec((1, D), lambda i: (0, 0)),
        ],
        out_specs=pl.BlockSpec((BN, D), lambda i: (i, 0)),
        out_shape=jax.ShapeDtypeStruct((N, D), jnp.float32),
    )(p, h, W, b, Wr, br)


def _dense2_readout(p, h, W, b, Wr, br, Wgp, bgp, Wp1, bp1, Wp2p, bp2p):
    nblk = N // BN

    def body(p_ref, h_ref, w_ref, b_ref, wr_ref, br_ref,
             wg_ref, bg_ref, wp1_ref, bp1_ref, wp2_ref, bp2_ref,
             o_ref, s_acc, m_acc):
        i = pl.program_id(0)
        m = p_ref[0] + p_ref[1]
        new = _elu(jnp.dot(m, w_ref[...], preferred_element_type=jnp.float32)
                   + b_ref[...])
        res = _elu(jnp.dot(h_ref[...], wr_ref[...],
                           preferred_element_type=jnp.float32) + br_ref[...])
        h2 = new + res
        gate = jnp.dot(h2, wg_ref[...], preferred_element_type=jnp.float32) \
            + bg_ref[...]
        w = 1.0 / (1.0 + jnp.exp(-gate[:, 0:1]))
        blk_sum = jnp.sum(w * h2, axis=0, keepdims=True)
        blk_max = jnp.max(h2, axis=0, keepdims=True)

        @pl.when(i == 0)
        def _():
            s_acc[...] = blk_sum
            m_acc[...] = blk_max

        @pl.when(i > 0)
        def _():
            s_acc[...] = s_acc[...] + blk_sum
            m_acc[...] = jnp.maximum(m_acc[...], blk_max)

        @pl.when(i == nblk - 1)
        def _():
            g = jnp.concatenate([s_acc[...], m_acc[...]], axis=1)
            z = jnp.maximum(
                jnp.dot(g, wp1_ref[...], preferred_element_type=jnp.float32)
                + bp1_ref[...], 0.0)
            z = z * (1.0 / jnp.sqrt(1.0 + 1e-5))
            o_ref[...] = jnp.dot(z, wp2_ref[...],
                                 preferred_element_type=jnp.float32) \
                + bp2_ref[...]

    return pl.pallas_call(
        body,
        grid=(nblk,),
        in_specs=[
            pl.BlockSpec((NC, BN, D), lambda i: (0, i, 0)),
            pl.BlockSpec((BN, D), lambda i: (i, 0)),
            pl.BlockSpec((D, D), lambda i: (0, 0)),
            pl.BlockSpec((1, D), lambda i: (0, 0)),
            pl.BlockSpec((D, D), lambda i: (0, 0)),
            pl.BlockSpec((1, D), lambda i: (0, 0)),
            pl.BlockSpec((D, D), lambda i: (0, 0)),
            pl.BlockSpec((1, D), lambda i: (0, 0)),
            pl.BlockSpec((2 * D, D), lambda i: (0, 0)),
            pl.BlockSpec((1, D), lambda i: (0, 0)),
            pl.BlockSpec((D, D), lambda i: (0, 0)),
            pl.BlockSpec((1, D), lambda i: (0, 0)),
        ],
        out_specs=pl.BlockSpec((1, D), lambda i: (0, 0)),
        out_shape=jax.ShapeDtypeStruct((1, D), jnp.float32),
        scratch_shapes=[
            pltpu.VMEM((1, D), jnp.float32),
            pltpu.VMEM((1, D), jnp.float32),
        ],
    )(p, h, W, b, Wr, br, Wgp, bgp, Wp1, bp1, Wp2p, bp2p)


def kernel(x, edge_index, W1, b1, Wr1, br1, W2, b2, Wr2, br2,
           Wg, bg, Wp1, bp1, Wp2, bp2):
    src3 = edge_index[0].reshape(NW, NCHUNK, CH)
    dst3 = edge_index[1].reshape(NW, NCHUNK, CH)
    zeros = jnp.zeros((NP, D), jnp.float32)

    Wgp = jnp.pad(Wg, ((0, 0), (0, D - Wg.shape[1])))
    bgp = jnp.pad(bg, (0, D - bg.shape[0])).reshape(1, D)
    Wp2p = jnp.pad(Wp2, ((0, 0), (0, D - Wp2.shape[1])))
    bp2p = jnp.pad(bp2, (0, D - bp2.shape[0])).reshape(1, D)

    b1r = b1.reshape(1, D)
    br1r = br1.reshape(1, D)
    b2r = b2.reshape(1, D)
    br2r = br2.reshape(1, D)
    bp1r = bp1.reshape(1, D)

    p1 = _seg_sum_partials(x, src3, dst3, zeros)
    h1 = _dense_layer(p1, x, W1, b1r, Wr1, br1r)
    p2 = _seg_sum_partials(h1, src3, dst3, zeros)
    out = _dense2_readout(p2, h1, W2, b2r, Wr2, br2r,
                          Wgp, bgp, Wp1, bp1r, Wp2p, bp2p)
    return out[:, :12]
